# baseline (speedup 1.0000x reference)
import jax
import jax.numpy as jnp
from jax.experimental import pallas as pl
from jax.experimental.pallas import tpu as pltpu

M = 65536
B = 8192
D = 128
BM = 4096
NB = M // BM


def _over(i, ptr):
    return ((i * BM - ptr) % M) < B


def _q_row(i, ptr):
    sp = ptr // BM
    neighbour = jnp.where(i * BM >= ptr, (sp - 1) % NB, (sp + B // BM) % NB)
    return jnp.where(_over(i, ptr), neighbour, i)


def _f_row(i, ptr):
    j0 = (i * BM - ptr) % M
    return jnp.minimum(j0 // BM, B // BM - 1)


def _q_index(kk):
    def index(k, i, ptr_ref):
        ptr = ptr_ref[0]
        return (jnp.where(k == kk, _q_row(i, ptr), _q_row(0, ptr)), 0)
    return index


def _f_index(kk):
    def index(k, i, ptr_ref):
        ptr = ptr_ref[0]
        return (jnp.where(k == kk, _f_row(i, ptr), _f_row(0, ptr)), 0)
    return index


def _out_index(k, i, ptr_ref):
    return (k, i, 0)


def _body(ptr_ref, q1, q2, q3, q4, q5, q6, f1, f2, f3, f4, f5, f6, out_ref):
    k = pl.program_id(0)
    i = pl.program_id(1)
    over = _over(i, ptr_ref[0])
    qs = (q1, q2, q3, q4, q5, q6)
    fs = (f1, f2, f3, f4, f5, f6)
    for kk in range(6):
        @pl.when(jnp.logical_and(k == kk, over))
        def _(kk=kk):
            out_ref[0, :, :] = fs[kk][:, :]

        @pl.when(jnp.logical_and(k == kk, jnp.logical_not(over)))
        def _(kk=kk):
            out_ref[0, :, :] = qs[kk][:, :]


@jax.jit
def kernel(p1_queue, r1_queue, p2_queue, r2_queue, p3_queue, r3_queue,
           feat_p1, feat_r1, feat_p2, feat_r2, feat_p3, feat_r3, ptr):
    ptr_arr = jnp.asarray(ptr, jnp.int32).reshape((1,))
    in_specs = ([pl.BlockSpec((BM, D), _q_index(kk)) for kk in range(6)]
                + [pl.BlockSpec((BM, D), _f_index(kk)) for kk in range(6)])
    grid_spec = pltpu.PrefetchScalarGridSpec(
        num_scalar_prefetch=1,
        grid=(6, NB),
        in_specs=in_specs,
        out_specs=pl.BlockSpec((1, BM, D), _out_index),
    )
    return pl.pallas_call(
        _body,
        grid_spec=grid_spec,
        out_shape=jax.ShapeDtypeStruct((6, M, D), jnp.float32),
        compiler_params=pltpu.CompilerParams(
            dimension_semantics=("arbitrary", "arbitrary"),
            vmem_limit_bytes=62 * 1024 * 1024,
        ),
    )(ptr_arr,
      p1_queue, r1_queue, p2_queue, r2_queue, p3_queue, r3_queue,
      feat_p1, feat_r1, feat_p2, feat_r2, feat_p3, feat_r3)


# speedup vs baseline: 1.2142x; 1.2142x over previous
import functools

import jax
import jax.numpy as jnp
from jax.experimental import pallas as pl
from jax.experimental.pallas import tpu as pltpu

M = 65536
B = 8192
D = 128
BM = 1024


def _body(ptr_ref, q1, q2, q3, q4, q5, q6, f1, f2, f3, f4, f5, f6, out_ref):
    i = pl.program_id(0)
    over = ((i * BM - ptr_ref[0]) % M) < B
    qs = (q1, q2, q3, q4, q5, q6)
    fs = (f1, f2, f3, f4, f5, f6)

    @pl.when(over)
    def _():
        for k in range(6):
            out_ref[k, :, :] = fs[k][:, :]

    @pl.when(jnp.logical_not(over))
    def _():
        for k in range(6):
            out_ref[k, :, :] = qs[k][:, :]


def _q_index(i, ptr_ref):
    nb = M // BM
    over = ((i * BM - ptr_ref[0]) % M) < B
    sp = ptr_ref[0] // BM
    neighbour = jnp.where(i * BM >= ptr_ref[0],
                          (sp - 1) % nb,
                          (sp + B // BM) % nb)
    return (jnp.where(over, neighbour, i), 0)


def _f_index(i, ptr_ref):
    j0 = (i * BM - ptr_ref[0]) % M
    return (jnp.minimum(j0 // BM, B // BM - 1), 0)


def _out_index(i, ptr_ref):
    return (0, i, 0)


@jax.jit
def kernel(p1_queue, r1_queue, p2_queue, r2_queue, p3_queue, r3_queue,
           feat_p1, feat_r1, feat_p2, feat_r2, feat_p3, feat_r3, ptr):
    ptr_arr = jnp.asarray(ptr, jnp.int32).reshape((1,))
    q_spec = pl.BlockSpec((BM, D), _q_index)
    f_spec = pl.BlockSpec((BM, D), _f_index)
    out_spec = pl.BlockSpec((6, BM, D), _out_index)
    grid_spec = pltpu.PrefetchScalarGridSpec(
        num_scalar_prefetch=1,
        grid=(M // BM,),
        in_specs=[q_spec] * 6 + [f_spec] * 6,
        out_specs=out_spec,
    )
    return pl.pallas_call(
        _body,
        grid_spec=grid_spec,
        out_shape=jax.ShapeDtypeStruct((6, M, D), jnp.float32),
        compiler_params=pltpu.CompilerParams(
            dimension_semantics=("arbitrary",),
        ),
    )(ptr_arr,
      p1_queue, r1_queue, p2_queue, r2_queue, p3_queue, r3_queue,
      feat_p1, feat_r1, feat_p2, feat_r2, feat_p3, feat_r3)


# speedup vs baseline: 1.2587x; 1.0367x over previous
import functools

import jax
import jax.numpy as jnp
from jax.experimental import pallas as pl
from jax.experimental.pallas import tpu as pltpu

M = 65536
B = 8192
D = 128
BM = 2048


def _body(ptr_ref, q1, q2, q3, q4, q5, q6, f1, f2, f3, f4, f5, f6, out_ref):
    i = pl.program_id(0)
    over = ((i * BM - ptr_ref[0]) % M) < B
    qs = (q1, q2, q3, q4, q5, q6)
    fs = (f1, f2, f3, f4, f5, f6)

    @pl.when(over)
    def _():
        for k in range(6):
            out_ref[k, :, :] = fs[k][:, :]

    @pl.when(jnp.logical_not(over))
    def _():
        for k in range(6):
            out_ref[k, :, :] = qs[k][:, :]


def _q_index(i, ptr_ref):
    nb = M // BM
    over = ((i * BM - ptr_ref[0]) % M) < B
    sp = ptr_ref[0] // BM
    neighbour = jnp.where(i * BM >= ptr_ref[0],
                          (sp - 1) % nb,
                          (sp + B // BM) % nb)
    return (jnp.where(over, neighbour, i), 0)


def _f_index(i, ptr_ref):
    j0 = (i * BM - ptr_ref[0]) % M
    return (jnp.minimum(j0 // BM, B // BM - 1), 0)


def _out_index(i, ptr_ref):
    return (0, i, 0)


@jax.jit
def kernel(p1_queue, r1_queue, p2_queue, r2_queue, p3_queue, r3_queue,
           feat_p1, feat_r1, feat_p2, feat_r2, feat_p3, feat_r3, ptr):
    ptr_arr = jnp.asarray(ptr, jnp.int32).reshape((1,))
    q_spec = pl.BlockSpec((BM, D), _q_index)
    f_spec = pl.BlockSpec((BM, D), _f_index)
    out_spec = pl.BlockSpec((6, BM, D), _out_index)
    grid_spec = pltpu.PrefetchScalarGridSpec(
        num_scalar_prefetch=1,
        grid=(M // BM,),
        in_specs=[q_spec] * 6 + [f_spec] * 6,
        out_specs=out_spec,
    )
    return pl.pallas_call(
        _body,
        grid_spec=grid_spec,
        out_shape=jax.ShapeDtypeStruct((6, M, D), jnp.float32),
        compiler_params=pltpu.CompilerParams(
            dimension_semantics=("arbitrary",),
        ),
    )(ptr_arr,
      p1_queue, r1_queue, p2_queue, r2_queue, p3_queue, r3_queue,
      feat_p1, feat_r1, feat_p2, feat_r2, feat_p3, feat_r3)
